# baseline (speedup 1.0000x reference)
import jax
import jax.numpy as jnp
from jax import lax
from jax.experimental import pallas as pl
from jax.experimental.pallas import tpu as pltpu
from jax.experimental.pallas import tpu_sc as plsc

_H = 64
_NC, _NS = 2, 16
_EB = 64
_G = 16
_STEP = _NS * _G * _EB
_Q = 4
_FQ = _H // _Q
_ACC = 100008

_TYPES = ['connection', 'contact_candidate', 'edge', 'face', 'part']
_NN = {'part': 10000, 'face': 100000, 'edge': 100000,
       'contact_candidate': 50000, 'connection': 10000}
_RELS = [('part', 'face'), ('face', 'face'), ('edge', 'face'),
         ('face', 'contact_candidate'), ('contact_candidate', 'connection')]
_ECNT = [100000, 800000, 200000, 100000, 50000]
_M = [min(_NN[s], _NN[d]) for s, d in _RELS]


def _round_up(x, m):
    return (x + m - 1) // m * m


def _bm_for(n):
    return {10000: 2000, 50000: 5000, 100000: 4000}[n]


def _enc(x, w, b):
    n, k = x.shape
    bm = _bm_for(n)

    def body(xr, wr, br, yr):
        yr[...] = jnp.maximum(
            jnp.dot(xr[...], wr[...], preferred_element_type=jnp.float32)
            + br[...], 0.0)

    return pl.pallas_call(
        body, grid=(n // bm,),
        in_specs=[pl.BlockSpec((bm, k), lambda i: (i, 0)),
                  pl.BlockSpec((k, _H), lambda i: (0, 0)),
                  pl.BlockSpec((1, _H), lambda i: (0, 0))],
        out_specs=pl.BlockSpec((bm, _H), lambda i: (i, 0)),
        out_shape=jax.ShapeDtypeStruct((n, _H), jnp.float32),
    )(x, w, b.reshape(1, _H))


def _msg(e, w):
    n = e.shape[0]
    bm = _bm_for(n)

    def body(er, wr, yr):
        yr[...] = jnp.maximum(
            jnp.dot(er[...], wr[...], preferred_element_type=jnp.float32), 0.0)

    return pl.pallas_call(
        body, grid=(n // bm,),
        in_specs=[pl.BlockSpec((bm, _H), lambda i: (i, 0)),
                  pl.BlockSpec((_H, _H), lambda i: (0, 0))],
        out_specs=pl.BlockSpec((bm, _H), lambda i: (i, 0)),
        out_shape=jax.ShapeDtypeStruct((n, _H), jnp.float32),
    )(e, w)


def _agg_specs(n, bm):
    nb = n // bm
    return [pl.BlockSpec((bm, _FQ), lambda i, q=q: (q * nb + i, 0))
            for q in range(_Q)]


def _upd(e, a4, w1, w2, b):
    n = e.shape[0]
    bm = _bm_for(n)

    def body(er, a0r, a1r, a2r, a3r, w1r, w2r, br, yr):
        h = (jnp.dot(er[...], w1r[...], preferred_element_type=jnp.float32)
             + br[...])
        w2v = w2r[...]
        for q, ar in enumerate((a0r, a1r, a2r, a3r)):
            h = h + jnp.dot(ar[...], w2v[q * _FQ:(q + 1) * _FQ, :],
                            preferred_element_type=jnp.float32)
        yr[...] = jnp.maximum(h, 0.0)

    return pl.pallas_call(
        body, grid=(n // bm,),
        in_specs=[pl.BlockSpec((bm, _H), lambda i: (i, 0))]
        + _agg_specs(n, bm)
        + [pl.BlockSpec((_H, _H), lambda i: (0, 0)),
           pl.BlockSpec((_H, _H), lambda i: (0, 0)),
           pl.BlockSpec((1, _H), lambda i: (0, 0))],
        out_specs=pl.BlockSpec((bm, _H), lambda i: (i, 0)),
        out_shape=jax.ShapeDtypeStruct((n, _H), jnp.float32),
    )(e, a4, a4, a4, a4, w1, w2, b.reshape(1, _H))


def _upd_head(e, a4, w1, w2, b, wh, bh):
    n = e.shape[0]
    bm = _bm_for(n)

    def body(er, a0r, a1r, a2r, a3r, w1r, w2r, br, whr, bhr, yr):
        h = (jnp.dot(er[...], w1r[...], preferred_element_type=jnp.float32)
             + br[...])
        w2v = w2r[...]
        for q, ar in enumerate((a0r, a1r, a2r, a3r)):
            h = h + jnp.dot(ar[...], w2v[q * _FQ:(q + 1) * _FQ, :],
                            preferred_element_type=jnp.float32)
        h = jnp.maximum(h, 0.0)
        yr[...] = jnp.dot(h, whr[...],
                          preferred_element_type=jnp.float32) + bhr[...]

    return pl.pallas_call(
        body, grid=(n // bm,),
        in_specs=[pl.BlockSpec((bm, _H), lambda i: (i, 0))]
        + _agg_specs(n, bm)
        + [pl.BlockSpec((_H, _H), lambda i: (0, 0)),
           pl.BlockSpec((_H, _H), lambda i: (0, 0)),
           pl.BlockSpec((1, _H), lambda i: (0, 0)),
           pl.BlockSpec((_H, 1), lambda i: (0, 0)),
           pl.BlockSpec((1, 1), lambda i: (0, 0))],
        out_specs=pl.BlockSpec((bm, 1), lambda i: (i, 0)),
        out_shape=jax.ShapeDtypeStruct((n, 1), jnp.float32),
    )(e, a4, a4, a4, a4, w1, w2, b.reshape(1, _H), wh, bh.reshape(1, 1))


def _passes_for(active):
    passes = []
    dsts = []
    for dst, cores in (('face', (0, 1, 0, 1)),
                       ('contact_candidate', (0, 0, 0, 0)),
                       ('connection', (1, 1, 1, 1))):
        rels_d = [r for r in active if _RELS[r][1] == dst]
        if not rels_d:
            continue
        dsts.append(dst)
        for q in range(_Q):
            passes.append(dict(core=cores[q], dst=dst, q=q, rels=rels_d))
    return passes, dsts


def _make_agg(active, epads):
    passes, dsts = _passes_for(active)
    nr = len(active)
    mesh = plsc.VectorSubcoreMesh(core_axis_name="c", subcore_axis_name="s",
                                  num_cores=_NC, num_subcores=_NS)

    def body(*refs):
        zs = dict(zip(active, refs[:nr]))
        sds = dict(zip(active, refs[nr:2 * nr]))
        zrows = refs[2 * nr]
        outs = dict(zip(dsts, refs[2 * nr + 1:2 * nr + 1 + len(dsts)]))
        acc, sdbuf, gbuf, rbuf, zbuf, sem = refs[2 * nr + 1 + len(dsts):]

        cid = lax.axis_index("c")
        tid = lax.axis_index("s")
        pltpu.sync_copy(zrows, zbuf)

        for p in passes:
            nd = _NN[p['dst']]
            q = p['q']

            @pl.when(cid == p['core'])
            def _run(p=p, nd=nd, q=q):
                zcnt0 = (nd // _NS) // 8 * 8
                zrem = nd - (_NS - 1) * zcnt0

                def _zero(off, cnt):
                    def zero_step(k, c):
                        pltpu.sync_copy(zbuf,
                                        acc.at[pl.ds(off + k * 256, 256)])
                        return c
                    lax.fori_loop(0, cnt // 256, zero_step, 0)
                    tail = cnt % 256
                    if tail:
                        pltpu.sync_copy(
                            zbuf.at[pl.ds(0, tail)],
                            acc.at[pl.ds(off + (cnt // 256) * 256, tail)])

                @pl.when(tid < _NS - 1)
                def _zero_main():
                    _zero(tid * zcnt0, zcnt0)

                @pl.when(tid == _NS - 1)
                def _zero_tail():
                    _zero((_NS - 1) * zcnt0, zrem)
                plsc.subcore_barrier()

                for r in p['rels']:
                    share = (epads[r] // _EB) // _NS
                    tb = tid * share

                    def scan_step(j, c, r=r, q=q, tb=tb):
                        row = tb + j * _G
                        pltpu.sync_copy(sds[r].at[pl.ds(row, _G)], sdbuf)
                        for g in range(_G):
                            for i in range(_EB // 16):
                                s = sdbuf[g, 0, pl.ds(i * 16, 16)]
                                gbuf[g, pl.ds(i * 16, 16)] = s * _Q + q
                        descs = [pltpu.async_copy(
                            zs[r].at[gbuf.at[g]], rbuf.at[g], sem)
                            for g in range(_G)]
                        for g in range(_G):
                            descs[g].wait()
                            pltpu.sync_copy(
                                rbuf.at[g], acc.at[sdbuf.at[g, 1]], add=True)
                        return c
                    lax.fori_loop(0, share // _G, scan_step, 0)
                plsc.subcore_barrier()

                cnt0 = (nd // _NS) // 8 * 8
                rem = nd - (_NS - 1) * cnt0
                out = outs[p['dst']]

                @pl.when(tid < _NS - 1)
                def _copy_main():
                    pltpu.sync_copy(
                        acc.at[pl.ds(tid * cnt0, cnt0)],
                        out.at[pl.ds(q * nd + tid * cnt0, cnt0)])

                @pl.when(tid == _NS - 1)
                def _copy_tail():
                    pltpu.sync_copy(
                        acc.at[pl.ds((_NS - 1) * cnt0, rem)],
                        out.at[pl.ds(q * nd + (_NS - 1) * cnt0, rem)])
                plsc.subcore_barrier()

    out_type = [jax.ShapeDtypeStruct((_Q * _NN[d], _FQ), jnp.float32)
                for d in dsts]
    kern = pl.kernel(
        body, out_type=out_type, mesh=mesh,
        compiler_params=pltpu.CompilerParams(use_tc_tiling_on_sc=False),
        scratch_types=[
            pltpu.VMEM_SHARED((_ACC, _FQ), jnp.float32),
            pltpu.VMEM((_G, 2, _EB), jnp.int32),
            pltpu.VMEM((_G, _EB), jnp.int32),
            pltpu.VMEM((_G, _EB, _FQ), jnp.float32),
            pltpu.VMEM((256, _FQ), jnp.float32),
            pltpu.SemaphoreType.DMA,
        ])
    return kern, dsts


def kernel(x_connection, x_contact_candidate, x_edge, x_face, x_part,
           ei_0, ei_1, ei_2, ei_3, ei_4,
           Wenc_connection, benc_connection,
           Wenc_contact_candidate, benc_contact_candidate,
           Wenc_edge, benc_edge,
           Wenc_face, benc_face,
           Wenc_part, benc_part,
           Wmsg, Wupd, bupd,
           Whead_conn, bhead_conn,
           Whead_face, bhead_face):
    xs = {'connection': x_connection, 'contact_candidate': x_contact_candidate,
          'edge': x_edge, 'face': x_face, 'part': x_part}
    wenc = {'connection': (Wenc_connection, benc_connection),
            'contact_candidate': (Wenc_contact_candidate,
                                  benc_contact_candidate),
            'edge': (Wenc_edge, benc_edge),
            'face': (Wenc_face, benc_face),
            'part': (Wenc_part, benc_part)}
    eis = [ei_0, ei_1, ei_2, ei_3, ei_4]

    epads, srcs, dsts_i = [], [], []
    for r, ei in enumerate(eis):
        e = _ECNT[r]
        epad = _round_up(e, _STEP)
        s = jnp.concatenate([ei[0], jnp.zeros((epad - e,), jnp.int32)])
        d = jnp.concatenate([ei[1], jnp.full((epad - e,),
                                             _NN[_RELS[r][1]], jnp.int32)])
        epads.append(epad)
        srcs.append(jnp.concatenate([s.reshape(-1, 1, _EB),
                                     d.reshape(-1, 1, _EB)], axis=1))
    zrows = jnp.zeros((256, _FQ), jnp.float32)

    emb = {t: _enc(xs[t], wenc[t][0], wenc[t][1]) for t in _TYPES}

    for l in range(3):
        active = [0, 1, 2, 4] if l == 2 else [0, 1, 2, 3, 4]
        z = {r: _msg(emb[_RELS[r][0]], Wmsg[l, r]) for r in active}
        agg_kern, agg_dsts = _make_agg(active, epads)
        args = ([z[r].reshape(_Q * z[r].shape[0], _FQ) for r in active]
                + [srcs[r] for r in active]
                + [zrows])
        aggs = dict(zip(agg_dsts, agg_kern(*args)))

        upd_types = ['connection', 'face'] if l == 2 else _TYPES
        new = {}
        for t in upd_types:
            ti = _TYPES.index(t)
            w1, w2 = Wupd[l, ti, :_H], Wupd[l, ti, _H:]
            b = bupd[l, ti]
            if l == 2:
                wh, bh = ((Whead_conn, bhead_conn) if t == 'connection'
                          else (Whead_face, bhead_face))
                new[t] = _upd_head(emb[t], aggs[t], w1, w2, b, wh, bh)
            elif t in aggs:
                new[t] = _upd(emb[t], aggs[t], w1, w2, b)
            else:
                new[t] = _enc(emb[t], w1, b)
        emb = {t: new.get(t, emb[t]) for t in _TYPES}

    return jnp.concatenate([emb['connection'], emb['face']], axis=0)


# speedup vs baseline: 1.2823x; 1.2823x over previous
import jax
import jax.numpy as jnp
from jax import lax
from jax.experimental import pallas as pl
from jax.experimental.pallas import tpu as pltpu
from jax.experimental.pallas import tpu_sc as plsc

_H = 64
_NC, _NS = 2, 16
_EB = 64
_G = 8
_STEP = _NS * _G * _EB
_Q = 4
_FQ = _H // _Q
_ACC = 100008

_TYPES = ['connection', 'contact_candidate', 'edge', 'face', 'part']
_NN = {'part': 10000, 'face': 100000, 'edge': 100000,
       'contact_candidate': 50000, 'connection': 10000}
_RELS = [('part', 'face'), ('face', 'face'), ('edge', 'face'),
         ('face', 'contact_candidate'), ('contact_candidate', 'connection')]
_ECNT = [100000, 800000, 200000, 100000, 50000]
_M = [min(_NN[s], _NN[d]) for s, d in _RELS]


def _round_up(x, m):
    return (x + m - 1) // m * m


def _bm_for(n):
    return {10000: 2000, 50000: 5000, 100000: 4000}[n]


def _enc(x, w, b):
    n, k = x.shape
    bm = _bm_for(n)

    def body(xr, wr, br, yr):
        yr[...] = jnp.maximum(
            jnp.dot(xr[...], wr[...], preferred_element_type=jnp.float32)
            + br[...], 0.0)

    return pl.pallas_call(
        body, grid=(n // bm,),
        in_specs=[pl.BlockSpec((bm, k), lambda i: (i, 0)),
                  pl.BlockSpec((k, _H), lambda i: (0, 0)),
                  pl.BlockSpec((1, _H), lambda i: (0, 0))],
        out_specs=pl.BlockSpec((bm, _H), lambda i: (i, 0)),
        out_shape=jax.ShapeDtypeStruct((n, _H), jnp.float32),
    )(x, w, b.reshape(1, _H))


def _msg(e, w):
    n = e.shape[0]
    bm = _bm_for(n)

    def body(er, wr, yr):
        yr[...] = jnp.maximum(
            jnp.dot(er[...], wr[...], preferred_element_type=jnp.float32), 0.0)

    return pl.pallas_call(
        body, grid=(n // bm,),
        in_specs=[pl.BlockSpec((bm, _H), lambda i: (i, 0)),
                  pl.BlockSpec((_H, _H), lambda i: (0, 0))],
        out_specs=pl.BlockSpec((bm, _H), lambda i: (i, 0)),
        out_shape=jax.ShapeDtypeStruct((n, _H), jnp.float32),
    )(e, w)


def _agg_specs(n, bm):
    nb = n // bm
    return [pl.BlockSpec((bm, _FQ), lambda i, q=q: (q * nb + i, 0))
            for q in range(_Q)]


def _upd(e, a4, w1, w2, b):
    n = e.shape[0]
    bm = _bm_for(n)

    def body(er, a0r, a1r, a2r, a3r, w1r, w2r, br, yr):
        h = (jnp.dot(er[...], w1r[...], preferred_element_type=jnp.float32)
             + br[...])
        w2v = w2r[...]
        for q, ar in enumerate((a0r, a1r, a2r, a3r)):
            h = h + jnp.dot(ar[...], w2v[q * _FQ:(q + 1) * _FQ, :],
                            preferred_element_type=jnp.float32)
        yr[...] = jnp.maximum(h, 0.0)

    return pl.pallas_call(
        body, grid=(n // bm,),
        in_specs=[pl.BlockSpec((bm, _H), lambda i: (i, 0))]
        + _agg_specs(n, bm)
        + [pl.BlockSpec((_H, _H), lambda i: (0, 0)),
           pl.BlockSpec((_H, _H), lambda i: (0, 0)),
           pl.BlockSpec((1, _H), lambda i: (0, 0))],
        out_specs=pl.BlockSpec((bm, _H), lambda i: (i, 0)),
        out_shape=jax.ShapeDtypeStruct((n, _H), jnp.float32),
    )(e, a4, a4, a4, a4, w1, w2, b.reshape(1, _H))


def _upd_head(e, a4, w1, w2, b, wh, bh):
    n = e.shape[0]
    bm = _bm_for(n)

    def body(er, a0r, a1r, a2r, a3r, w1r, w2r, br, whr, bhr, yr):
        h = (jnp.dot(er[...], w1r[...], preferred_element_type=jnp.float32)
             + br[...])
        w2v = w2r[...]
        for q, ar in enumerate((a0r, a1r, a2r, a3r)):
            h = h + jnp.dot(ar[...], w2v[q * _FQ:(q + 1) * _FQ, :],
                            preferred_element_type=jnp.float32)
        h = jnp.maximum(h, 0.0)
        yr[...] = jnp.dot(h, whr[...],
                          preferred_element_type=jnp.float32) + bhr[...]

    return pl.pallas_call(
        body, grid=(n // bm,),
        in_specs=[pl.BlockSpec((bm, _H), lambda i: (i, 0))]
        + _agg_specs(n, bm)
        + [pl.BlockSpec((_H, _H), lambda i: (0, 0)),
           pl.BlockSpec((_H, _H), lambda i: (0, 0)),
           pl.BlockSpec((1, _H), lambda i: (0, 0)),
           pl.BlockSpec((_H, 1), lambda i: (0, 0)),
           pl.BlockSpec((1, 1), lambda i: (0, 0))],
        out_specs=pl.BlockSpec((bm, 1), lambda i: (i, 0)),
        out_shape=jax.ShapeDtypeStruct((n, 1), jnp.float32),
    )(e, a4, a4, a4, a4, w1, w2, b.reshape(1, _H), wh, bh.reshape(1, 1))


def _passes_for(active):
    passes = []
    dsts = []
    for dst, cores in (('face', (0, 1, 0, 1)),
                       ('contact_candidate', (0, 0, 0, 0)),
                       ('connection', (1, 1, 1, 1))):
        rels_d = [r for r in active if _RELS[r][1] == dst]
        if not rels_d:
            continue
        dsts.append(dst)
        for q in range(_Q):
            passes.append(dict(core=cores[q], dst=dst, q=q, rels=rels_d))
    return passes, dsts


def _make_agg(active, epads):
    passes, dsts = _passes_for(active)
    nr = len(active)
    mesh = plsc.VectorSubcoreMesh(core_axis_name="c", subcore_axis_name="s",
                                  num_cores=_NC, num_subcores=_NS)

    def body(*refs):
        zs = dict(zip(active, refs[:nr]))
        sds = dict(zip(active, refs[nr:2 * nr]))
        zrows = refs[2 * nr]
        outs = dict(zip(dsts, refs[2 * nr + 1:2 * nr + 1 + len(dsts)]))
        acc, sdbuf, gbuf, rbuf, zbuf, sem, sem2 = refs[2 * nr + 1
                                                       + len(dsts):]

        cid = lax.axis_index("c")
        tid = lax.axis_index("s")
        pltpu.sync_copy(zrows, zbuf)

        for p in passes:
            nd = _NN[p['dst']]
            q = p['q']

            @pl.when(cid == p['core'])
            def _run(p=p, nd=nd, q=q):
                zcnt0 = (nd // _NS) // 8 * 8
                zrem = nd - (_NS - 1) * zcnt0

                def _zero(off, cnt):
                    def zero_step(k, c):
                        pltpu.sync_copy(zbuf,
                                        acc.at[pl.ds(off + k * 256, 256)])
                        return c
                    lax.fori_loop(0, cnt // 256, zero_step, 0)
                    tail = cnt % 256
                    if tail:
                        pltpu.sync_copy(
                            zbuf.at[pl.ds(0, tail)],
                            acc.at[pl.ds(off + (cnt // 256) * 256, tail)])

                @pl.when(tid < _NS - 1)
                def _zero_main():
                    _zero(tid * zcnt0, zcnt0)

                @pl.when(tid == _NS - 1)
                def _zero_tail():
                    _zero((_NS - 1) * zcnt0, zrem)
                plsc.subcore_barrier()

                for r in p['rels']:
                    share = (epads[r] // _EB) // _NS
                    tb = tid * share
                    nsteps = share // _G

                    def proc(j, pb, r=r, q=q, tb=tb, nsteps=nsteps):
                        row = tb + j * _G
                        pltpu.make_async_copy(
                            sds[r].at[pl.ds(row, _G)], sdbuf.at[pb],
                            sem2).wait()
                        nxt = jnp.minimum(j + 1, nsteps - 1)
                        pltpu.async_copy(
                            sds[r].at[pl.ds(tb + nxt * _G, _G)],
                            sdbuf.at[1 - pb], sem2)
                        for g in range(_G):
                            for i in range(_EB // 16):
                                s = sdbuf[pb, g, 0, pl.ds(i * 16, 16)]
                                gbuf[g, pl.ds(i * 16, 16)] = s * _Q + q
                        descs = [pltpu.async_copy(
                            zs[r].at[gbuf.at[g]], rbuf.at[g], sem)
                            for g in range(_G)]
                        for g in range(_G):
                            descs[g].wait()
                            pltpu.sync_copy(
                                rbuf.at[g], acc.at[sdbuf.at[pb, g, 1]],
                                add=True)

                    def scan_step(j, c):
                        @pl.when(lax.rem(j, 2) == 0)
                        def _even():
                            proc(j, 0)

                        @pl.when(lax.rem(j, 2) == 1)
                        def _odd():
                            proc(j, 1)
                        return c

                    pltpu.async_copy(sds[r].at[pl.ds(tb, _G)],
                                     sdbuf.at[0], sem2)
                    lax.fori_loop(0, nsteps, scan_step, 0)
                    pltpu.make_async_copy(
                        sds[r].at[pl.ds(tb, _G)],
                        sdbuf.at[nsteps % 2], sem2).wait()
                plsc.subcore_barrier()

                cnt0 = (nd // _NS) // 8 * 8
                rem = nd - (_NS - 1) * cnt0
                out = outs[p['dst']]

                @pl.when(tid < _NS - 1)
                def _copy_main():
                    pltpu.sync_copy(
                        acc.at[pl.ds(tid * cnt0, cnt0)],
                        out.at[pl.ds(q * nd + tid * cnt0, cnt0)])

                @pl.when(tid == _NS - 1)
                def _copy_tail():
                    pltpu.sync_copy(
                        acc.at[pl.ds((_NS - 1) * cnt0, rem)],
                        out.at[pl.ds(q * nd + (_NS - 1) * cnt0, rem)])
                plsc.subcore_barrier()

    out_type = [jax.ShapeDtypeStruct((_Q * _NN[d], _FQ), jnp.float32)
                for d in dsts]
    kern = pl.kernel(
        body, out_type=out_type, mesh=mesh,
        compiler_params=pltpu.CompilerParams(use_tc_tiling_on_sc=False),
        scratch_types=[
            pltpu.VMEM_SHARED((_ACC, _FQ), jnp.float32),
            pltpu.VMEM((2, _G, 2, _EB), jnp.int32),
            pltpu.VMEM((_G, _EB), jnp.int32),
            pltpu.VMEM((_G, _EB, _FQ), jnp.float32),
            pltpu.VMEM((256, _FQ), jnp.float32),
            pltpu.SemaphoreType.DMA,
            pltpu.SemaphoreType.DMA,
        ])
    return kern, dsts


def kernel(x_connection, x_contact_candidate, x_edge, x_face, x_part,
           ei_0, ei_1, ei_2, ei_3, ei_4,
           Wenc_connection, benc_connection,
           Wenc_contact_candidate, benc_contact_candidate,
           Wenc_edge, benc_edge,
           Wenc_face, benc_face,
           Wenc_part, benc_part,
           Wmsg, Wupd, bupd,
           Whead_conn, bhead_conn,
           Whead_face, bhead_face):
    xs = {'connection': x_connection, 'contact_candidate': x_contact_candidate,
          'edge': x_edge, 'face': x_face, 'part': x_part}
    wenc = {'connection': (Wenc_connection, benc_connection),
            'contact_candidate': (Wenc_contact_candidate,
                                  benc_contact_candidate),
            'edge': (Wenc_edge, benc_edge),
            'face': (Wenc_face, benc_face),
            'part': (Wenc_part, benc_part)}
    eis = [ei_0, ei_1, ei_2, ei_3, ei_4]

    epads, srcs, dsts_i = [], [], []
    for r, ei in enumerate(eis):
        e = _ECNT[r]
        epad = _round_up(e, _STEP)
        s = jnp.concatenate([ei[0], jnp.zeros((epad - e,), jnp.int32)])
        d = jnp.concatenate([ei[1], jnp.full((epad - e,),
                                             _NN[_RELS[r][1]], jnp.int32)])
        epads.append(epad)
        srcs.append(jnp.concatenate([s.reshape(-1, 1, _EB),
                                     d.reshape(-1, 1, _EB)], axis=1))
    zrows = jnp.zeros((256, _FQ), jnp.float32)

    emb = {t: _enc(xs[t], wenc[t][0], wenc[t][1]) for t in _TYPES}

    for l in range(3):
        active = [0, 1, 2, 4] if l == 2 else [0, 1, 2, 3, 4]
        z = {r: _msg(emb[_RELS[r][0]], Wmsg[l, r]) for r in active}
        agg_kern, agg_dsts = _make_agg(active, epads)
        args = ([z[r].reshape(_Q * z[r].shape[0], _FQ) for r in active]
                + [srcs[r] for r in active]
                + [zrows])
        aggs = dict(zip(agg_dsts, agg_kern(*args)))

        upd_types = ['connection', 'face'] if l == 2 else _TYPES
        new = {}
        for t in upd_types:
            ti = _TYPES.index(t)
            w1, w2 = Wupd[l, ti, :_H], Wupd[l, ti, _H:]
            b = bupd[l, ti]
            if l == 2:
                wh, bh = ((Whead_conn, bhead_conn) if t == 'connection'
                          else (Whead_face, bhead_face))
                new[t] = _upd_head(emb[t], aggs[t], w1, w2, b, wh, bh)
            elif t in aggs:
                new[t] = _upd(emb[t], aggs[t], w1, w2, b)
            else:
                new[t] = _enc(emb[t], w1, b)
        emb = {t: new.get(t, emb[t]) for t in _TYPES}

    return jnp.concatenate([emb['connection'], emb['face']], axis=0)
